# baseline (speedup 1.0000x reference)
import jax
import jax.numpy as jnp
from jax import lax
from jax.experimental import pallas as pl
from jax.experimental.pallas import tpu as pltpu
from jax.experimental.pallas import tpu_sc as plsc

N = 10000
E = 160000
D = 512
H = 256
HH = H // 2

NC = 2
NS = 16
LANES = 16

R = 10240
BUCKETS = NS
BROWS = R // BUCKETS
E_PAD = 163840
EPT_DEG = E_PAD // (NC * NS)
CHUNK = 128
NCH_S = EPT_DEG // CHUNK

_MESH = plsc.VectorSubcoreMesh(
    core_axis_name="c", subcore_axis_name="s", num_cores=NC, num_subcores=NS
)

_SC_PARAMS = pltpu.CompilerParams(needs_layout_passes=False)


def _sc_degrees_body(srcd_hbm, dstd_hbm, dego_hbm, degi_hbm, bcnt_hbm,
                     src_v, dst_v, dego_v, degi_v, bcnt_v):
    cid = lax.axis_index("c")
    sid = lax.axis_index("s")
    wid = sid * NC + cid

    pltpu.sync_copy(srcd_hbm.at[wid], src_v)
    pltpu.sync_copy(dstd_hbm.at[wid], dst_v)

    zeros16 = jnp.zeros((LANES,), jnp.float32)

    def zero_body(i, _):
        dego_v[pl.ds(i * LANES, LANES)] = zeros16
        degi_v[pl.ds(i * LANES, LANES)] = zeros16
        return _

    lax.fori_loop(0, R // LANES, zero_body, None)
    bcnt_v[...] = jnp.zeros((BUCKETS,), jnp.int32)

    ones16 = jnp.ones((LANES,), jnp.float32)
    ones_i = jnp.ones((LANES,), jnp.int32)

    def hist_body(i, _):
        s_idx = src_v[pl.ds(i * LANES, LANES)]
        plsc.addupdate_scatter(dego_v, [s_idx], ones16)
        d_idx = dst_v[pl.ds(i * LANES, LANES)]
        plsc.addupdate_scatter(degi_v, [d_idx], ones16)
        plsc.addupdate_scatter(bcnt_v, [d_idx // BROWS], ones_i)
        return _

    lax.fori_loop(0, EPT_DEG // LANES, hist_body, None)

    pltpu.sync_copy(dego_v, dego_hbm.at[wid])
    pltpu.sync_copy(degi_v, degi_hbm.at[wid])
    pltpu.sync_copy(bcnt_v, bcnt_hbm.at[wid])


_sc_degrees = pl.kernel(
    _sc_degrees_body,
    out_type=(
        jax.ShapeDtypeStruct((NC * NS, R), jnp.float32),
        jax.ShapeDtypeStruct((NC * NS, R), jnp.float32),
        jax.ShapeDtypeStruct((NC * NS, BUCKETS), jnp.int32),
    ),
    mesh=_MESH,
    scratch_types=[
        pltpu.VMEM((EPT_DEG,), jnp.int32),
        pltpu.VMEM((EPT_DEG,), jnp.int32),
        pltpu.VMEM((R,), jnp.float32),
        pltpu.VMEM((R,), jnp.float32),
        pltpu.VMEM((BUCKETS,), jnp.int32),
    ],
    compiler_params=_SC_PARAMS,
)


def _sc_sort_body(srcd_hbm, dstd_hbm, base_hbm, srcs_hbm, dsts_hbm,
                  src_v, dst_v, base_v, cnt_v, posb,
                  sem_a, sem_b, sem_c, sem_d):
    cid = lax.axis_index("c")
    sid = lax.axis_index("s")
    wid = sid * NC + cid

    pltpu.sync_copy(srcd_hbm.at[wid], src_v)
    pltpu.sync_copy(dstd_hbm.at[wid], dst_v)
    pltpu.sync_copy(base_hbm.at[wid], base_v)
    cnt_v[...] = jnp.zeros((BUCKETS,), jnp.int32)

    ones_i = jnp.ones((LANES,), jnp.int32)

    def do_chunk(c, prow, s1, s2):
        for k in range(CHUNK // LANES):
            d = dst_v[pl.ds(c * CHUNK + k * LANES, LANES)]
            b = d // BROWS
            cnt_g = plsc.load_gather(cnt_v, [b])
            base_g = plsc.load_gather(base_v, [b])
            rank = jnp.zeros((LANES,), jnp.int32)
            for bb in range(BUCKETS):
                m = b == bb
                cs = plsc.cumsum(m.astype(jnp.int32))
                rank = jnp.where(m, cs - 1, rank)
            posb[prow, pl.ds(k * LANES, LANES)] = base_g + cnt_g + rank
            plsc.addupdate_scatter(cnt_v, [b], ones_i)
        pltpu.async_copy(src_v.at[pl.ds(c * CHUNK, CHUNK)],
                         srcs_hbm.at[posb.at[prow]], s1)
        pltpu.async_copy(dst_v.at[pl.ds(c * CHUNK, CHUNK)],
                         dsts_hbm.at[posb.at[prow]], s2)

    def wait_sc(prow, s):
        pltpu.make_async_copy(src_v.at[pl.ds(0, CHUNK)],
                              srcs_hbm.at[posb.at[prow]], s).wait()

    def pair_body(p, _):
        do_chunk(2 * p, 0, sem_a, sem_b)
        do_chunk(2 * p + 1, 1, sem_c, sem_d)
        wait_sc(0, sem_a)
        wait_sc(0, sem_b)
        wait_sc(1, sem_c)
        wait_sc(1, sem_d)
        return _

    lax.fori_loop(0, NCH_S // 2, pair_body, None)


_sc_sort = pl.kernel(
    _sc_sort_body,
    out_type=(
        jax.ShapeDtypeStruct((E_PAD,), jnp.int32),
        jax.ShapeDtypeStruct((E_PAD,), jnp.int32),
    ),
    mesh=_MESH,
    scratch_types=[
        pltpu.VMEM((EPT_DEG,), jnp.int32),
        pltpu.VMEM((EPT_DEG,), jnp.int32),
        pltpu.VMEM((BUCKETS,), jnp.int32),
        pltpu.VMEM((BUCKETS,), jnp.int32),
        pltpu.VMEM((2, CHUNK), jnp.int32),
        pltpu.SemaphoreType.DMA,
        pltpu.SemaphoreType.DMA,
        pltpu.SemaphoreType.DMA,
        pltpu.SemaphoreType.DMA,
    ],
    compiler_params=_SC_PARAMS,
)


def _sc_prop_body(table_hbm, srcs_hbm, dsts_hbm, offs_hbm, agg_hbm,
                  src_c, dst_c, rows0, rows1, acc_t, offs_v, sem0, sem1):
    cid = lax.axis_index("c")
    sid = lax.axis_index("s")

    pltpu.sync_copy(offs_hbm, offs_v)
    sid_vec = jnp.zeros((LANES,), jnp.int32) + sid
    lo = plsc.load_gather(offs_v, [jnp.zeros((LANES,), jnp.int32),
                                   sid_vec])[0]
    hi = plsc.load_gather(offs_v, [jnp.zeros((LANES,), jnp.int32),
                                   sid_vec + 1])[0]
    lo_al = (lo // CHUNK) * CHUNK
    n_chunks = (hi - lo_al + CHUNK - 1) // CHUNK

    zeros16 = jnp.zeros((LANES,), jnp.float32)

    def zb(i, _):
        r = i // (HH // LANES)
        k = i % (HH // LANES)
        acc_t[r, pl.ds(k * LANES, LANES)] = zeros16
        return _

    lax.fori_loop(0, BROWS * (HH // LANES), zb, None)

    col_off = cid * R

    def stage(c, buf):
        start = lo_al + c * CHUNK
        pltpu.sync_copy(srcs_hbm.at[pl.ds(start, CHUNK)], src_c.at[buf])
        pltpu.sync_copy(dsts_hbm.at[pl.ds(start, CHUNK)], dst_c.at[buf])
        for k in range(CHUNK // LANES):
            sl = pl.ds(k * LANES, LANES)
            src_c[buf, sl] = src_c[buf, sl] + col_off

    def issue(buf, rows, sem):
        pltpu.async_copy(table_hbm.at[src_c.at[buf]], rows, sem)

    def wait_g(rows, sem):
        pltpu.make_async_copy(table_hbm.at[src_c.at[0]], rows, sem).wait()

    base_row = sid * BROWS

    def accum(buf, rows):
        def rb(g, _):
            dvec = dst_c[buf, pl.ds(g * LANES, LANES)] - base_row
            for lane in range(LANES):
                dl = dvec[lane]
                r2 = g * LANES + lane

                @pl.when((dl >= 0) & (dl < BROWS))
                def _():
                    for k in range(HH // LANES):
                        sl = pl.ds(k * LANES, LANES)
                        acc_t[dl, sl] = acc_t[dl, sl] + rows[r2, sl]

            return _

        lax.fori_loop(0, CHUNK // LANES, rb, None)

    @pl.when(n_chunks > 0)
    def _():
        stage(0, 0)
        issue(0, rows0, sem0)

        def pair_body(p, _):
            c1 = 2 * p + 1

            @pl.when(c1 < n_chunks)
            def _():
                stage(c1, 1)
                issue(1, rows1, sem1)

            wait_g(rows0, sem0)
            accum(0, rows0)

            @pl.when(c1 + 1 < n_chunks)
            def _():
                stage(c1 + 1, 0)
                issue(0, rows0, sem0)

            @pl.when(c1 < n_chunks)
            def _():
                wait_g(rows1, sem1)
                accum(1, rows1)

            return _

        lax.fori_loop(0, (n_chunks + 1) // 2, pair_body, None)

    pltpu.sync_copy(acc_t, agg_hbm.at[pl.ds(col_off + base_row, BROWS)])


_sc_propagate = pl.kernel(
    _sc_prop_body,
    out_type=jax.ShapeDtypeStruct((NC * R, HH), jnp.float32),
    mesh=_MESH,
    scratch_types=[
        pltpu.VMEM((2, CHUNK), jnp.int32),
        pltpu.VMEM((2, CHUNK), jnp.int32),
        pltpu.VMEM((CHUNK, HH), jnp.float32),
        pltpu.VMEM((CHUNK, HH), jnp.float32),
        pltpu.VMEM((BROWS, HH), jnp.float32),
        pltpu.VMEM((1, 2 * BUCKETS), jnp.int32),
        pltpu.SemaphoreType.DMA,
        pltpu.SemaphoreType.DMA,
    ],
    compiler_params=_SC_PARAMS,
)


_BM = 1024
_GRID = R // _BM
_NWB = NC * NS * BUCKETS


def _dot(a, b):
    return jnp.dot(a, b, preferred_element_type=jnp.float32,
                   precision=lax.Precision.HIGHEST)


def _tc_finalize_body(dego_ref, degi_ref, bcnt_ref,
                      dinv_out_ref, dinv_in_ref, base_ref, offs_ref):
    do = jnp.maximum(jnp.sum(dego_ref[...], axis=0), 1.0)
    di = jnp.maximum(jnp.sum(degi_ref[...], axis=0), 1.0)
    dinv_out_ref[...] = lax.rsqrt(do)[:, None]
    dinv_in_ref[...] = lax.rsqrt(di)[:, None]

    cnt_f = bcnt_ref[...].astype(jnp.float32)
    colsum = jnp.sum(cnt_f, axis=0, keepdims=True)
    bi = lax.broadcasted_iota(jnp.int32, (BUCKETS, BUCKETS), 0)
    bj = lax.broadcasted_iota(jnp.int32, (BUCKETS, BUCKETS), 1)
    mask_b = (bi < bj).astype(jnp.float32)
    off_row = _dot(colsum, mask_b)
    nw = NC * NS
    wi = lax.broadcasted_iota(jnp.int32, (nw, nw), 0)
    wj = lax.broadcasted_iota(jnp.int32, (nw, nw), 1)
    mask_w = (wj < wi).astype(jnp.float32)
    base = _dot(mask_w, cnt_f) + off_row
    base_ref[...] = base.astype(jnp.int32)
    offs_ref[...] = jnp.concatenate(
        [off_row.astype(jnp.int32),
         jnp.full((1, BUCKETS), E_PAD, jnp.int32)], axis=1)


def _tc_finalize(dego_p, degi_p, bcnt_p):
    return pl.pallas_call(
        _tc_finalize_body,
        out_shape=(
            jax.ShapeDtypeStruct((R, 1), jnp.float32),
            jax.ShapeDtypeStruct((R, 1), jnp.float32),
            jax.ShapeDtypeStruct((NC * NS, BUCKETS), jnp.int32),
            jax.ShapeDtypeStruct((1, 2 * BUCKETS), jnp.int32),
        ),
    )(dego_p, degi_p, bcnt_p)


def _tc_y1_body(x_ref, w_ref, dinv_ref, out_ref):
    y = _dot(x_ref[...], w_ref[...]) * dinv_ref[...]
    out_ref[0, :, :] = y[:, :HH]
    out_ref[1, :, :] = y[:, HH:]


def _tc_y1(feats_p, W_gc1, dinv_out):
    return pl.pallas_call(
        _tc_y1_body,
        grid=(_GRID,),
        in_specs=[
            pl.BlockSpec((_BM, D), lambda r: (r, 0)),
            pl.BlockSpec((D, H), lambda r: (0, 0)),
            pl.BlockSpec((_BM, 1), lambda r: (r, 0)),
        ],
        out_specs=pl.BlockSpec((NC, _BM, HH), lambda r: (0, r, 0)),
        out_shape=jax.ShapeDtypeStruct((NC, R, HH), jnp.float32),
    )(feats_p, W_gc1, dinv_out)


def _tc_mid_body(agg_ref, din_ref, dout_ref, b_ref, out_ref):
    x = jax.nn.relu(agg_ref[...] * din_ref[...][None] + b_ref[...])
    out_ref[...] = x * dout_ref[...][None]


def _tc_mid(agg1, dinv_in, dinv_out, b_gc1_2):
    return pl.pallas_call(
        _tc_mid_body,
        grid=(_GRID,),
        in_specs=[
            pl.BlockSpec((NC, _BM, HH), lambda r: (0, r, 0)),
            pl.BlockSpec((_BM, 1), lambda r: (r, 0)),
            pl.BlockSpec((_BM, 1), lambda r: (r, 0)),
            pl.BlockSpec((NC, 1, HH), lambda r: (0, 0, 0)),
        ],
        out_specs=pl.BlockSpec((NC, _BM, HH), lambda r: (0, r, 0)),
        out_shape=jax.ShapeDtypeStruct((NC, R, HH), jnp.float32),
    )(agg1, dinv_in, dinv_out, b_gc1_2)


def _tc_final_body(agg_ref, din_ref, feat_ref, wg2_ref, bg2_ref,
                   wm1_ref, bm1_ref, wm2_ref, bm2_ref, out_ref):
    a = jnp.concatenate([agg_ref[0], agg_ref[1]], axis=1) * din_ref[...]
    gcn = _dot(a, wg2_ref[...]) + bg2_ref[...] + feat_ref[...]
    m = jax.nn.relu(_dot(gcn, wm1_ref[...]) + bm1_ref[...])
    out_ref[...] = _dot(m, wm2_ref[...]) + bm2_ref[...] + gcn


def _tc_final(agg2, dinv_in, feats_p, W_gc2, b_gc2, W_m1, b_m1, W_m2, b_m2):
    return pl.pallas_call(
        _tc_final_body,
        grid=(_GRID,),
        in_specs=[
            pl.BlockSpec((NC, _BM, HH), lambda r: (0, r, 0)),
            pl.BlockSpec((_BM, 1), lambda r: (r, 0)),
            pl.BlockSpec((_BM, D), lambda r: (r, 0)),
            pl.BlockSpec((H, D), lambda r: (0, 0)),
            pl.BlockSpec((1, D), lambda r: (0, 0)),
            pl.BlockSpec((D, H), lambda r: (0, 0)),
            pl.BlockSpec((1, H), lambda r: (0, 0)),
            pl.BlockSpec((H, D), lambda r: (0, 0)),
            pl.BlockSpec((1, D), lambda r: (0, 0)),
        ],
        out_specs=pl.BlockSpec((_BM, D), lambda r: (r, 0)),
        out_shape=jax.ShapeDtypeStruct((R, D), jnp.float32),
    )(agg2, dinv_in, feats_p, W_gc2, b_gc2, W_m1, b_m1, W_m2, b_m2)


def kernel(features, edge_index, W_gc1, b_gc1, W_gc2, b_gc2,
           W_m1, b_m1, W_m2, b_m2):
    src = edge_index[0]
    dst = edge_index[1]
    pad = jnp.full((E_PAD - E,), N, dtype=jnp.int32)
    src_p = jnp.concatenate([src, pad])
    dst_p = jnp.concatenate([dst, pad])

    srcd = src_p.reshape(NC * NS, EPT_DEG)
    dstd = dst_p.reshape(NC * NS, EPT_DEG)

    feats_p = jnp.pad(features, ((0, R - N), (0, 0)))

    dego_p, degi_p, bcnt_p = _sc_degrees(srcd, dstd)
    dinv_out, dinv_in, base32, offs = _tc_finalize(dego_p, degi_p, bcnt_p)
    srcs, dsts = _sc_sort(srcd, dstd, base32)

    y1 = _tc_y1(feats_p, W_gc1, dinv_out)
    agg1 = _sc_propagate(y1.reshape(NC * R, HH), srcs, dsts, offs)
    h2 = _tc_mid(agg1.reshape(NC, R, HH), dinv_in, dinv_out,
                 b_gc1.reshape(NC, 1, HH))
    agg2 = _sc_propagate(h2.reshape(NC * R, HH), srcs, dsts, offs)
    out_p = _tc_final(agg2.reshape(NC, R, HH), dinv_in, feats_p,
                      W_gc2, b_gc2.reshape(1, D), W_m1, b_m1.reshape(1, H),
                      W_m2, b_m2.reshape(1, D))
    return out_p[:N]


# speedup vs baseline: 1.0017x; 1.0017x over previous
import jax
import jax.numpy as jnp
from jax import lax
from jax.experimental import pallas as pl
from jax.experimental.pallas import tpu as pltpu
from jax.experimental.pallas import tpu_sc as plsc

N = 10000
E = 160000
D = 512
H = 256
HH = H // 2

NC = 2
NS = 16
LANES = 16

R = 10240
BUCKETS = NS
BROWS = R // BUCKETS
E_PAD = 163840
EPT_DEG = E_PAD // (NC * NS)
CHUNK = 128
NCH_S = EPT_DEG // CHUNK

_MESH = plsc.VectorSubcoreMesh(
    core_axis_name="c", subcore_axis_name="s", num_cores=NC, num_subcores=NS
)

_SC_PARAMS = pltpu.CompilerParams(needs_layout_passes=False)


def _sc_degrees_body(srcd_hbm, dstd_hbm, dego_hbm, degi_hbm, bcnt_hbm,
                     src_v, dst_v, dego_v, degi_v, bcnt_v):
    cid = lax.axis_index("c")
    sid = lax.axis_index("s")
    wid = sid * NC + cid

    pltpu.sync_copy(srcd_hbm.at[wid], src_v)
    pltpu.sync_copy(dstd_hbm.at[wid], dst_v)

    zeros16 = jnp.zeros((LANES,), jnp.float32)

    def zero_body(i, _):
        dego_v[pl.ds(i * LANES, LANES)] = zeros16
        degi_v[pl.ds(i * LANES, LANES)] = zeros16
        return _

    lax.fori_loop(0, R // LANES, zero_body, None)
    bcnt_v[...] = jnp.zeros((BUCKETS,), jnp.int32)

    ones16 = jnp.ones((LANES,), jnp.float32)
    ones_i = jnp.ones((LANES,), jnp.int32)

    def hist_body(i, _):
        s_idx = src_v[pl.ds(i * LANES, LANES)]
        plsc.addupdate_scatter(dego_v, [s_idx], ones16)
        d_idx = dst_v[pl.ds(i * LANES, LANES)]
        plsc.addupdate_scatter(degi_v, [d_idx], ones16)
        plsc.addupdate_scatter(bcnt_v, [d_idx // BROWS], ones_i)
        return _

    lax.fori_loop(0, EPT_DEG // LANES, hist_body, None)

    pltpu.sync_copy(dego_v, dego_hbm.at[wid])
    pltpu.sync_copy(degi_v, degi_hbm.at[wid])
    pltpu.sync_copy(bcnt_v, bcnt_hbm.at[wid])


_sc_degrees = pl.kernel(
    _sc_degrees_body,
    out_type=(
        jax.ShapeDtypeStruct((NC * NS, R), jnp.float32),
        jax.ShapeDtypeStruct((NC * NS, R), jnp.float32),
        jax.ShapeDtypeStruct((NC * NS, BUCKETS), jnp.int32),
    ),
    mesh=_MESH,
    scratch_types=[
        pltpu.VMEM((EPT_DEG,), jnp.int32),
        pltpu.VMEM((EPT_DEG,), jnp.int32),
        pltpu.VMEM((R,), jnp.float32),
        pltpu.VMEM((R,), jnp.float32),
        pltpu.VMEM((BUCKETS,), jnp.int32),
    ],
    compiler_params=_SC_PARAMS,
)


def _sc_sort_body(srcd_hbm, dstd_hbm, base_hbm, srcs_hbm, dsts_hbm,
                  src_v, dst_v, base_v, cnt_v, posb,
                  sem_a, sem_b, sem_c, sem_d):
    cid = lax.axis_index("c")
    sid = lax.axis_index("s")
    wid = sid * NC + cid

    pltpu.sync_copy(srcd_hbm.at[wid], src_v)
    pltpu.sync_copy(dstd_hbm.at[wid], dst_v)
    pltpu.sync_copy(base_hbm.at[wid], base_v)
    cnt_v[...] = jnp.zeros((BUCKETS,), jnp.int32)

    ones_i = jnp.ones((LANES,), jnp.int32)

    def do_chunk(c, prow, s1):
        for k in range(CHUNK // LANES):
            d = dst_v[pl.ds(c * CHUNK + k * LANES, LANES)]
            b = d // BROWS
            cnt_g = plsc.load_gather(cnt_v, [b])
            base_g = plsc.load_gather(base_v, [b])
            rank, _ = plsc.scan_count(b)
            posb[prow, pl.ds(k * LANES, LANES)] = base_g + cnt_g + rank - 1
            plsc.addupdate_scatter(cnt_v, [b], ones_i)
        pltpu.async_copy(src_v.at[pl.ds(c * CHUNK, CHUNK)],
                         srcs_hbm.at[posb.at[prow]], s1)
        pltpu.async_copy(dst_v.at[pl.ds(c * CHUNK, CHUNK)],
                         dsts_hbm.at[posb.at[prow]], s1)

    def wait_sc(prow, s):
        pltpu.make_async_copy(src_v.at[pl.ds(0, CHUNK)],
                              srcs_hbm.at[posb.at[prow]], s).wait()

    sems = (sem_a, sem_b, sem_c, sem_d)

    def quad_body(p, _):
        for q in range(4):
            @pl.when(p > 0)
            def _():
                wait_sc(q, sems[q])
                wait_sc(q, sems[q])

            do_chunk(4 * p + q, q, sems[q])
        return _

    lax.fori_loop(0, NCH_S // 4, quad_body, None)
    for q in range(4):
        wait_sc(q, sems[q])
        wait_sc(q, sems[q])


_sc_sort = pl.kernel(
    _sc_sort_body,
    out_type=(
        jax.ShapeDtypeStruct((E_PAD,), jnp.int32),
        jax.ShapeDtypeStruct((E_PAD,), jnp.int32),
    ),
    mesh=_MESH,
    scratch_types=[
        pltpu.VMEM((EPT_DEG,), jnp.int32),
        pltpu.VMEM((EPT_DEG,), jnp.int32),
        pltpu.VMEM((BUCKETS,), jnp.int32),
        pltpu.VMEM((BUCKETS,), jnp.int32),
        pltpu.VMEM((4, CHUNK), jnp.int32),
        pltpu.SemaphoreType.DMA,
        pltpu.SemaphoreType.DMA,
        pltpu.SemaphoreType.DMA,
        pltpu.SemaphoreType.DMA,
    ],
    compiler_params=_SC_PARAMS,
)


def _sc_prop_body(table_hbm, srcs_hbm, dsts_hbm, offs_hbm, agg_hbm,
                  src_c, dst_c, dl_buf, rows0, rows1, acc_t, offs_v,
                  sem0, sem1):
    cid = lax.axis_index("c")
    sid = lax.axis_index("s")

    pltpu.sync_copy(offs_hbm, offs_v)
    sid_vec = jnp.zeros((LANES,), jnp.int32) + sid
    lo = plsc.load_gather(offs_v, [jnp.zeros((LANES,), jnp.int32),
                                   sid_vec])[0]
    hi = plsc.load_gather(offs_v, [jnp.zeros((LANES,), jnp.int32),
                                   sid_vec + 1])[0]
    lo_al = (lo // CHUNK) * CHUNK
    n_chunks = (hi - lo_al + CHUNK - 1) // CHUNK

    zeros16 = jnp.zeros((LANES,), jnp.float32)

    def zb(i, _):
        r = i // (HH // LANES)
        k = i % (HH // LANES)
        acc_t[r, pl.ds(k * LANES, LANES)] = zeros16
        return _

    lax.fori_loop(0, (BROWS + 1) * (HH // LANES), zb, None)

    col_off = cid * R

    def stage(c, buf):
        start = lo_al + c * CHUNK
        pltpu.sync_copy(srcs_hbm.at[pl.ds(start, CHUNK)], src_c.at[buf])
        pltpu.sync_copy(dsts_hbm.at[pl.ds(start, CHUNK)], dst_c.at[buf])
        for k in range(CHUNK // LANES):
            sl = pl.ds(k * LANES, LANES)
            src_c[buf, sl] = src_c[buf, sl] + col_off

    def issue(buf, rows, sem):
        pltpu.async_copy(table_hbm.at[src_c.at[buf]], rows, sem)

    def wait_g(rows, sem):
        pltpu.make_async_copy(table_hbm.at[src_c.at[0]], rows, sem).wait()

    base_row = sid * BROWS

    def accum(buf, rows):
        def rb(g, _):
            dl0 = dst_c[buf, pl.ds(g * LANES, LANES)] - base_row
            dlv = jnp.where((dl0 >= 0) & (dl0 < BROWS), dl0, BROWS)
            for lane in range(LANES):
                dl = dlv[lane]
                r2 = g * LANES + lane
                for k in range(HH // LANES):
                    sl = pl.ds(k * LANES, LANES)
                    acc_t[dl, sl] = acc_t[dl, sl] + rows[r2, sl]
            return _

        lax.fori_loop(0, CHUNK // LANES, rb, None)

    @pl.when(n_chunks > 0)
    def _():
        stage(0, 0)
        issue(0, rows0, sem0)

        def pair_body(p, _):
            c1 = 2 * p + 1

            @pl.when(c1 < n_chunks)
            def _():
                stage(c1, 1)
                issue(1, rows1, sem1)

            wait_g(rows0, sem0)
            accum(0, rows0)

            @pl.when(c1 + 1 < n_chunks)
            def _():
                stage(c1 + 1, 0)
                issue(0, rows0, sem0)

            @pl.when(c1 < n_chunks)
            def _():
                wait_g(rows1, sem1)
                accum(1, rows1)

            return _

        lax.fori_loop(0, (n_chunks + 1) // 2, pair_body, None)

    pltpu.sync_copy(acc_t.at[pl.ds(0, BROWS)],
                    agg_hbm.at[pl.ds(col_off + base_row, BROWS)])


_sc_propagate = pl.kernel(
    _sc_prop_body,
    out_type=jax.ShapeDtypeStruct((NC * R, HH), jnp.float32),
    mesh=_MESH,
    scratch_types=[
        pltpu.VMEM((2, CHUNK), jnp.int32),
        pltpu.VMEM((2, CHUNK), jnp.int32),
        pltpu.VMEM((2, CHUNK), jnp.int32),
        pltpu.VMEM((CHUNK, HH), jnp.float32),
        pltpu.VMEM((CHUNK, HH), jnp.float32),
        pltpu.VMEM((BROWS + 1, HH), jnp.float32),
        pltpu.VMEM((1, 2 * BUCKETS), jnp.int32),
        pltpu.SemaphoreType.DMA,
        pltpu.SemaphoreType.DMA,
    ],
    compiler_params=_SC_PARAMS,
)


_BM = 1024
_GRID = R // _BM
_NWB = NC * NS * BUCKETS


def _dot(a, b):
    return jnp.dot(a, b, preferred_element_type=jnp.float32,
                   precision=lax.Precision.HIGHEST)


def _tc_finalize_body(dego_ref, degi_ref, bcnt_ref,
                      dinv_out_ref, dinv_in_ref, base_ref, offs_ref):
    do = jnp.maximum(jnp.sum(dego_ref[...], axis=0), 1.0)
    di = jnp.maximum(jnp.sum(degi_ref[...], axis=0), 1.0)
    dinv_out_ref[...] = lax.rsqrt(do)[:, None]
    dinv_in_ref[...] = lax.rsqrt(di)[:, None]

    cnt_f = bcnt_ref[...].astype(jnp.float32)
    colsum = jnp.sum(cnt_f, axis=0, keepdims=True)
    bi = lax.broadcasted_iota(jnp.int32, (BUCKETS, BUCKETS), 0)
    bj = lax.broadcasted_iota(jnp.int32, (BUCKETS, BUCKETS), 1)
    mask_b = (bi < bj).astype(jnp.float32)
    off_row = _dot(colsum, mask_b)
    nw = NC * NS
    wi = lax.broadcasted_iota(jnp.int32, (nw, nw), 0)
    wj = lax.broadcasted_iota(jnp.int32, (nw, nw), 1)
    mask_w = (wj < wi).astype(jnp.float32)
    base = _dot(mask_w, cnt_f) + off_row
    base_ref[...] = base.astype(jnp.int32)
    offs_ref[...] = jnp.concatenate(
        [off_row.astype(jnp.int32),
         jnp.full((1, BUCKETS), E_PAD, jnp.int32)], axis=1)


def _tc_finalize(dego_p, degi_p, bcnt_p):
    return pl.pallas_call(
        _tc_finalize_body,
        out_shape=(
            jax.ShapeDtypeStruct((R, 1), jnp.float32),
            jax.ShapeDtypeStruct((R, 1), jnp.float32),
            jax.ShapeDtypeStruct((NC * NS, BUCKETS), jnp.int32),
            jax.ShapeDtypeStruct((1, 2 * BUCKETS), jnp.int32),
        ),
    )(dego_p, degi_p, bcnt_p)


def _tc_y1_body(x_ref, w_ref, dinv_ref, out_ref):
    y = _dot(x_ref[...], w_ref[...]) * dinv_ref[...]
    out_ref[0, :, :] = y[:, :HH]
    out_ref[1, :, :] = y[:, HH:]


def _tc_y1(feats_p, W_gc1, dinv_out):
    return pl.pallas_call(
        _tc_y1_body,
        grid=(_GRID,),
        in_specs=[
            pl.BlockSpec((_BM, D), lambda r: (r, 0)),
            pl.BlockSpec((D, H), lambda r: (0, 0)),
            pl.BlockSpec((_BM, 1), lambda r: (r, 0)),
        ],
        out_specs=pl.BlockSpec((NC, _BM, HH), lambda r: (0, r, 0)),
        out_shape=jax.ShapeDtypeStruct((NC, R, HH), jnp.float32),
    )(feats_p, W_gc1, dinv_out)


def _tc_mid_body(agg_ref, din_ref, dout_ref, b_ref, out_ref):
    x = jax.nn.relu(agg_ref[...] * din_ref[...][None] + b_ref[...])
    out_ref[...] = x * dout_ref[...][None]


def _tc_mid(agg1, dinv_in, dinv_out, b_gc1_2):
    return pl.pallas_call(
        _tc_mid_body,
        grid=(_GRID,),
        in_specs=[
            pl.BlockSpec((NC, _BM, HH), lambda r: (0, r, 0)),
            pl.BlockSpec((_BM, 1), lambda r: (r, 0)),
            pl.BlockSpec((_BM, 1), lambda r: (r, 0)),
            pl.BlockSpec((NC, 1, HH), lambda r: (0, 0, 0)),
        ],
        out_specs=pl.BlockSpec((NC, _BM, HH), lambda r: (0, r, 0)),
        out_shape=jax.ShapeDtypeStruct((NC, R, HH), jnp.float32),
    )(agg1, dinv_in, dinv_out, b_gc1_2)


def _tc_final_body(agg_ref, din_ref, feat_ref, wg2_ref, bg2_ref,
                   wm1_ref, bm1_ref, wm2_ref, bm2_ref, out_ref):
    a = jnp.concatenate([agg_ref[0], agg_ref[1]], axis=1) * din_ref[...]
    gcn = _dot(a, wg2_ref[...]) + bg2_ref[...] + feat_ref[...]
    m = jax.nn.relu(_dot(gcn, wm1_ref[...]) + bm1_ref[...])
    out_ref[...] = _dot(m, wm2_ref[...]) + bm2_ref[...] + gcn


def _tc_final(agg2, dinv_in, feats_p, W_gc2, b_gc2, W_m1, b_m1, W_m2, b_m2):
    return pl.pallas_call(
        _tc_final_body,
        grid=(_GRID,),
        in_specs=[
            pl.BlockSpec((NC, _BM, HH), lambda r: (0, r, 0)),
            pl.BlockSpec((_BM, 1), lambda r: (r, 0)),
            pl.BlockSpec((_BM, D), lambda r: (r, 0)),
            pl.BlockSpec((H, D), lambda r: (0, 0)),
            pl.BlockSpec((1, D), lambda r: (0, 0)),
            pl.BlockSpec((D, H), lambda r: (0, 0)),
            pl.BlockSpec((1, H), lambda r: (0, 0)),
            pl.BlockSpec((H, D), lambda r: (0, 0)),
            pl.BlockSpec((1, D), lambda r: (0, 0)),
        ],
        out_specs=pl.BlockSpec((_BM, D), lambda r: (r, 0)),
        out_shape=jax.ShapeDtypeStruct((R, D), jnp.float32),
    )(agg2, dinv_in, feats_p, W_gc2, b_gc2, W_m1, b_m1, W_m2, b_m2)


def kernel(features, edge_index, W_gc1, b_gc1, W_gc2, b_gc2,
           W_m1, b_m1, W_m2, b_m2):
    src = edge_index[0]
    dst = edge_index[1]
    pad = jnp.full((E_PAD - E,), N, dtype=jnp.int32)
    src_p = jnp.concatenate([src, pad])
    dst_p = jnp.concatenate([dst, pad])

    srcd = src_p.reshape(NC * NS, EPT_DEG)
    dstd = dst_p.reshape(NC * NS, EPT_DEG)

    feats_p = jnp.pad(features, ((0, R - N), (0, 0)))

    dego_p, degi_p, bcnt_p = _sc_degrees(srcd, dstd)
    dinv_out, dinv_in, base32, offs = _tc_finalize(dego_p, degi_p, bcnt_p)
    srcs, dsts = _sc_sort(srcd, dstd, base32)

    y1 = _tc_y1(feats_p, W_gc1, dinv_out)
    agg1 = _sc_propagate(y1.reshape(NC * R, HH), srcs, dsts, offs)
    h2 = _tc_mid(agg1.reshape(NC, R, HH), dinv_in, dinv_out,
                 b_gc1.reshape(NC, 1, HH))
    agg2 = _sc_propagate(h2.reshape(NC * R, HH), srcs, dsts, offs)
    out_p = _tc_final(agg2.reshape(NC, R, HH), dinv_in, feats_p,
                      W_gc2, b_gc2.reshape(1, D), W_m1, b_m1.reshape(1, H),
                      W_m2, b_m2.reshape(1, D))
    return out_p[:N]


# speedup vs baseline: 1.0729x; 1.0711x over previous
import jax
import jax.numpy as jnp
from jax import lax
from jax.experimental import pallas as pl
from jax.experimental.pallas import tpu as pltpu
from jax.experimental.pallas import tpu_sc as plsc

N = 10000
E = 160000
D = 512
H = 256
HH = H // 2

NC = 2
NS = 16
LANES = 16

R = 10240
BUCKETS = NS
BROWS = R // BUCKETS
E_PAD = 163840
EPT_DEG = E_PAD // (NC * NS)
CHUNK = 128
NCH_S = EPT_DEG // CHUNK

_MESH = plsc.VectorSubcoreMesh(
    core_axis_name="c", subcore_axis_name="s", num_cores=NC, num_subcores=NS
)

_SC_PARAMS = pltpu.CompilerParams(needs_layout_passes=False)


def _sc_degrees_body(srcd_hbm, dstd_hbm, dego_hbm, degi_hbm, bcnt_hbm,
                     src_v, dst_v, dego_v, degi_v, bcnt_v):
    cid = lax.axis_index("c")
    sid = lax.axis_index("s")
    wid = sid * NC + cid

    pltpu.sync_copy(srcd_hbm.at[wid], src_v)
    pltpu.sync_copy(dstd_hbm.at[wid], dst_v)

    zeros16 = jnp.zeros((LANES,), jnp.float32)

    def zero_body(i, _):
        dego_v[pl.ds(i * LANES, LANES)] = zeros16
        degi_v[pl.ds(i * LANES, LANES)] = zeros16
        return _

    lax.fori_loop(0, R // LANES, zero_body, None)
    bcnt_v[...] = jnp.zeros((BUCKETS,), jnp.int32)

    ones16 = jnp.ones((LANES,), jnp.float32)
    ones_i = jnp.ones((LANES,), jnp.int32)

    def hist_body(i, _):
        s_idx = src_v[pl.ds(i * LANES, LANES)]
        plsc.addupdate_scatter(dego_v, [s_idx], ones16)
        d_idx = dst_v[pl.ds(i * LANES, LANES)]
        plsc.addupdate_scatter(degi_v, [d_idx], ones16)
        plsc.addupdate_scatter(bcnt_v, [d_idx // BROWS], ones_i)
        return _

    lax.fori_loop(0, EPT_DEG // LANES, hist_body, None)

    pltpu.sync_copy(dego_v, dego_hbm.at[wid])
    pltpu.sync_copy(degi_v, degi_hbm.at[wid])
    pltpu.sync_copy(bcnt_v, bcnt_hbm.at[wid])


_sc_degrees = pl.kernel(
    _sc_degrees_body,
    out_type=(
        jax.ShapeDtypeStruct((NC * NS, R), jnp.float32),
        jax.ShapeDtypeStruct((NC * NS, R), jnp.float32),
        jax.ShapeDtypeStruct((NC * NS, BUCKETS), jnp.int32),
    ),
    mesh=_MESH,
    scratch_types=[
        pltpu.VMEM((EPT_DEG,), jnp.int32),
        pltpu.VMEM((EPT_DEG,), jnp.int32),
        pltpu.VMEM((R,), jnp.float32),
        pltpu.VMEM((R,), jnp.float32),
        pltpu.VMEM((BUCKETS,), jnp.int32),
    ],
    compiler_params=_SC_PARAMS,
)


def _sc_sort_body(srcd_hbm, dstd_hbm, base_hbm, srcs_hbm, dsts_hbm,
                  src_v, dst_v, base_v, cnt_v, posb,
                  sem_a, sem_b, sem_c, sem_d):
    cid = lax.axis_index("c")
    sid = lax.axis_index("s")
    wid = sid * NC + cid

    pltpu.sync_copy(srcd_hbm.at[wid], src_v)
    pltpu.sync_copy(dstd_hbm.at[wid], dst_v)
    pltpu.sync_copy(base_hbm.at[wid], base_v)
    cnt_v[...] = jnp.zeros((BUCKETS,), jnp.int32)

    ones_i = jnp.ones((LANES,), jnp.int32)

    def do_chunk(c, prow, s1):
        for k in range(CHUNK // LANES):
            d = dst_v[pl.ds(c * CHUNK + k * LANES, LANES)]
            b = d // BROWS
            cnt_g = plsc.load_gather(cnt_v, [b])
            base_g = plsc.load_gather(base_v, [b])
            rank, _ = plsc.scan_count(b)
            posb[prow, pl.ds(k * LANES, LANES)] = base_g + cnt_g + rank - 1
            plsc.addupdate_scatter(cnt_v, [b], ones_i)
        pltpu.async_copy(src_v.at[pl.ds(c * CHUNK, CHUNK)],
                         srcs_hbm.at[posb.at[prow]], s1)
        pltpu.async_copy(dst_v.at[pl.ds(c * CHUNK, CHUNK)],
                         dsts_hbm.at[posb.at[prow]], s1)

    def wait_sc(prow, s):
        pltpu.make_async_copy(src_v.at[pl.ds(0, CHUNK)],
                              srcs_hbm.at[posb.at[prow]], s).wait()

    sems = (sem_a, sem_b, sem_c, sem_d)

    def quad_body(p, _):
        for q in range(4):
            @pl.when(p > 0)
            def _():
                wait_sc(q, sems[q])
                wait_sc(q, sems[q])

            do_chunk(4 * p + q, q, sems[q])
        return _

    lax.fori_loop(0, NCH_S // 4, quad_body, None)
    for q in range(4):
        wait_sc(q, sems[q])
        wait_sc(q, sems[q])


_sc_sort = pl.kernel(
    _sc_sort_body,
    out_type=(
        jax.ShapeDtypeStruct((E_PAD,), jnp.int32),
        jax.ShapeDtypeStruct((E_PAD,), jnp.int32),
    ),
    mesh=_MESH,
    scratch_types=[
        pltpu.VMEM((EPT_DEG,), jnp.int32),
        pltpu.VMEM((EPT_DEG,), jnp.int32),
        pltpu.VMEM((BUCKETS,), jnp.int32),
        pltpu.VMEM((BUCKETS,), jnp.int32),
        pltpu.VMEM((4, CHUNK), jnp.int32),
        pltpu.SemaphoreType.DMA,
        pltpu.SemaphoreType.DMA,
        pltpu.SemaphoreType.DMA,
        pltpu.SemaphoreType.DMA,
    ],
    compiler_params=_SC_PARAMS,
)


def _sc_prop_body(table_hbm, srcs_hbm, dsts_hbm, offs_hbm, agg_hbm,
                  src_c, dst_c, dl_buf, rows0, rows1, acc_t, offs_v,
                  sem0, sem1):
    cid = lax.axis_index("c")
    sid = lax.axis_index("s")

    pltpu.sync_copy(offs_hbm, offs_v)
    sid_vec = jnp.zeros((LANES,), jnp.int32) + sid
    lo = plsc.load_gather(offs_v, [jnp.zeros((LANES,), jnp.int32),
                                   sid_vec])[0]
    hi = plsc.load_gather(offs_v, [jnp.zeros((LANES,), jnp.int32),
                                   sid_vec + 1])[0]
    lo_al = (lo // CHUNK) * CHUNK
    n_chunks = (hi - lo_al + CHUNK - 1) // CHUNK

    zeros16 = jnp.zeros((LANES,), jnp.float32)

    def zb(i, _):
        acc_t[pl.ds(i * LANES, LANES)] = zeros16
        return _

    lax.fori_loop(0, (BROWS + 1) * HH // LANES, zb, None)

    col_off = cid * R

    def stage(c, buf):
        start = lo_al + c * CHUNK
        pltpu.sync_copy(srcs_hbm.at[pl.ds(start, CHUNK)], src_c.at[buf])
        pltpu.sync_copy(dsts_hbm.at[pl.ds(start, CHUNK)], dst_c.at[buf])
        for k in range(CHUNK // LANES):
            sl = pl.ds(k * LANES, LANES)
            src_c[buf, sl] = src_c[buf, sl] + col_off

    def issue(buf, rows, sem):
        pltpu.async_copy(table_hbm.at[src_c.at[buf]], rows, sem)

    def wait_g(rows, sem):
        pltpu.make_async_copy(table_hbm.at[src_c.at[0]], rows, sem).wait()

    base_row = sid * BROWS

    iota16 = lax.iota(jnp.int32, LANES)

    def accum(buf, rows):
        def rb(g, _):
            dl0 = dst_c[buf, pl.ds(g * LANES, LANES)] - base_row
            dlv = jnp.where((dl0 >= 0) & (dl0 < BROWS), dl0, BROWS) * HH
            for lane in range(LANES):
                r2 = g * LANES + lane
                base_vec = iota16 + dlv[lane]
                for k in range(HH // LANES):
                    plsc.addupdate_scatter(
                        acc_t, [base_vec + (k * LANES)],
                        rows[r2, pl.ds(k * LANES, LANES)])
            return _

        lax.fori_loop(0, CHUNK // LANES, rb, None)

    @pl.when(n_chunks > 0)
    def _():
        stage(0, 0)
        issue(0, rows0, sem0)

        def pair_body(p, _):
            c1 = 2 * p + 1

            @pl.when(c1 < n_chunks)
            def _():
                stage(c1, 1)
                issue(1, rows1, sem1)

            wait_g(rows0, sem0)
            accum(0, rows0)

            @pl.when(c1 + 1 < n_chunks)
            def _():
                stage(c1 + 1, 0)
                issue(0, rows0, sem0)

            @pl.when(c1 < n_chunks)
            def _():
                wait_g(rows1, sem1)
                accum(1, rows1)

            return _

        lax.fori_loop(0, (n_chunks + 1) // 2, pair_body, None)

    pltpu.sync_copy(acc_t.at[pl.ds(0, BROWS * HH)],
                    agg_hbm.at[pl.ds((col_off + base_row) * HH, BROWS * HH)])


_sc_propagate = pl.kernel(
    _sc_prop_body,
    out_type=jax.ShapeDtypeStruct((NC * R * HH,), jnp.float32),
    mesh=_MESH,
    scratch_types=[
        pltpu.VMEM((2, CHUNK), jnp.int32),
        pltpu.VMEM((2, CHUNK), jnp.int32),
        pltpu.VMEM((2, CHUNK), jnp.int32),
        pltpu.VMEM((CHUNK, HH), jnp.float32),
        pltpu.VMEM((CHUNK, HH), jnp.float32),
        pltpu.VMEM(((BROWS + 1) * HH,), jnp.float32),
        pltpu.VMEM((1, 2 * BUCKETS), jnp.int32),
        pltpu.SemaphoreType.DMA,
        pltpu.SemaphoreType.DMA,
    ],
    compiler_params=_SC_PARAMS,
)


_BM = 1024
_GRID = R // _BM
_NWB = NC * NS * BUCKETS


def _dot(a, b):
    return jnp.dot(a, b, preferred_element_type=jnp.float32,
                   precision=lax.Precision.HIGHEST)


def _tc_finalize_body(dego_ref, degi_ref, bcnt_ref,
                      dinv_out_ref, dinv_in_ref, base_ref, offs_ref):
    do = jnp.maximum(jnp.sum(dego_ref[...], axis=0), 1.0)
    di = jnp.maximum(jnp.sum(degi_ref[...], axis=0), 1.0)
    dinv_out_ref[...] = lax.rsqrt(do)[:, None]
    dinv_in_ref[...] = lax.rsqrt(di)[:, None]

    cnt_f = bcnt_ref[...].astype(jnp.float32)
    colsum = jnp.sum(cnt_f, axis=0, keepdims=True)
    bi = lax.broadcasted_iota(jnp.int32, (BUCKETS, BUCKETS), 0)
    bj = lax.broadcasted_iota(jnp.int32, (BUCKETS, BUCKETS), 1)
    mask_b = (bi < bj).astype(jnp.float32)
    off_row = _dot(colsum, mask_b)
    nw = NC * NS
    wi = lax.broadcasted_iota(jnp.int32, (nw, nw), 0)
    wj = lax.broadcasted_iota(jnp.int32, (nw, nw), 1)
    mask_w = (wj < wi).astype(jnp.float32)
    base = _dot(mask_w, cnt_f) + off_row
    base_ref[...] = base.astype(jnp.int32)
    offs_ref[...] = jnp.concatenate(
        [off_row.astype(jnp.int32),
         jnp.full((1, BUCKETS), E_PAD, jnp.int32)], axis=1)


def _tc_finalize(dego_p, degi_p, bcnt_p):
    return pl.pallas_call(
        _tc_finalize_body,
        out_shape=(
            jax.ShapeDtypeStruct((R, 1), jnp.float32),
            jax.ShapeDtypeStruct((R, 1), jnp.float32),
            jax.ShapeDtypeStruct((NC * NS, BUCKETS), jnp.int32),
            jax.ShapeDtypeStruct((1, 2 * BUCKETS), jnp.int32),
        ),
    )(dego_p, degi_p, bcnt_p)


def _tc_y1_body(x_ref, w_ref, dinv_ref, out_ref):
    y = _dot(x_ref[...], w_ref[...]) * dinv_ref[...]
    out_ref[0, :, :] = y[:, :HH]
    out_ref[1, :, :] = y[:, HH:]


def _tc_y1(feats_p, W_gc1, dinv_out):
    return pl.pallas_call(
        _tc_y1_body,
        grid=(_GRID,),
        in_specs=[
            pl.BlockSpec((_BM, D), lambda r: (r, 0)),
            pl.BlockSpec((D, H), lambda r: (0, 0)),
            pl.BlockSpec((_BM, 1), lambda r: (r, 0)),
        ],
        out_specs=pl.BlockSpec((NC, _BM, HH), lambda r: (0, r, 0)),
        out_shape=jax.ShapeDtypeStruct((NC, R, HH), jnp.float32),
    )(feats_p, W_gc1, dinv_out)


def _tc_mid_body(agg_ref, din_ref, dout_ref, b_ref, out_ref):
    x = jax.nn.relu(agg_ref[...] * din_ref[...][None] + b_ref[...])
    out_ref[...] = x * dout_ref[...][None]


def _tc_mid(agg1, dinv_in, dinv_out, b_gc1_2):
    return pl.pallas_call(
        _tc_mid_body,
        grid=(_GRID,),
        in_specs=[
            pl.BlockSpec((NC, _BM, HH), lambda r: (0, r, 0)),
            pl.BlockSpec((_BM, 1), lambda r: (r, 0)),
            pl.BlockSpec((_BM, 1), lambda r: (r, 0)),
            pl.BlockSpec((NC, 1, HH), lambda r: (0, 0, 0)),
        ],
        out_specs=pl.BlockSpec((NC, _BM, HH), lambda r: (0, r, 0)),
        out_shape=jax.ShapeDtypeStruct((NC, R, HH), jnp.float32),
    )(agg1, dinv_in, dinv_out, b_gc1_2)


def _tc_final_body(agg_ref, din_ref, feat_ref, wg2_ref, bg2_ref,
                   wm1_ref, bm1_ref, wm2_ref, bm2_ref, out_ref):
    a = jnp.concatenate([agg_ref[0], agg_ref[1]], axis=1) * din_ref[...]
    gcn = _dot(a, wg2_ref[...]) + bg2_ref[...] + feat_ref[...]
    m = jax.nn.relu(_dot(gcn, wm1_ref[...]) + bm1_ref[...])
    out_ref[...] = _dot(m, wm2_ref[...]) + bm2_ref[...] + gcn


def _tc_final(agg2, dinv_in, feats_p, W_gc2, b_gc2, W_m1, b_m1, W_m2, b_m2):
    return pl.pallas_call(
        _tc_final_body,
        grid=(_GRID,),
        in_specs=[
            pl.BlockSpec((NC, _BM, HH), lambda r: (0, r, 0)),
            pl.BlockSpec((_BM, 1), lambda r: (r, 0)),
            pl.BlockSpec((_BM, D), lambda r: (r, 0)),
            pl.BlockSpec((H, D), lambda r: (0, 0)),
            pl.BlockSpec((1, D), lambda r: (0, 0)),
            pl.BlockSpec((D, H), lambda r: (0, 0)),
            pl.BlockSpec((1, H), lambda r: (0, 0)),
            pl.BlockSpec((H, D), lambda r: (0, 0)),
            pl.BlockSpec((1, D), lambda r: (0, 0)),
        ],
        out_specs=pl.BlockSpec((_BM, D), lambda r: (r, 0)),
        out_shape=jax.ShapeDtypeStruct((R, D), jnp.float32),
    )(agg2, dinv_in, feats_p, W_gc2, b_gc2, W_m1, b_m1, W_m2, b_m2)


def kernel(features, edge_index, W_gc1, b_gc1, W_gc2, b_gc2,
           W_m1, b_m1, W_m2, b_m2):
    src = edge_index[0]
    dst = edge_index[1]
    pad = jnp.full((E_PAD - E,), N, dtype=jnp.int32)
    src_p = jnp.concatenate([src, pad])
    dst_p = jnp.concatenate([dst, pad])

    srcd = src_p.reshape(NC * NS, EPT_DEG)
    dstd = dst_p.reshape(NC * NS, EPT_DEG)

    feats_p = jnp.pad(features, ((0, R - N), (0, 0)))

    dego_p, degi_p, bcnt_p = _sc_degrees(srcd, dstd)
    dinv_out, dinv_in, base32, offs = _tc_finalize(dego_p, degi_p, bcnt_p)
    srcs, dsts = _sc_sort(srcd, dstd, base32)

    y1 = _tc_y1(feats_p, W_gc1, dinv_out)
    agg1 = _sc_propagate(y1.reshape(NC * R, HH), srcs, dsts, offs)
    h2 = _tc_mid(agg1.reshape(NC, R, HH), dinv_in, dinv_out,
                 b_gc1.reshape(NC, 1, HH))
    agg2 = _sc_propagate(h2.reshape(NC * R, HH), srcs, dsts, offs)
    out_p = _tc_final(agg2.reshape(NC, R, HH), dinv_in, feats_p,
                      W_gc2, b_gc2.reshape(1, D), W_m1, b_m1.reshape(1, H),
                      W_m2, b_m2.reshape(1, D))
    return out_p[:N]


# speedup vs baseline: 2.1954x; 2.0461x over previous
import functools

import jax
import jax.numpy as jnp
from jax import lax
from jax.experimental import pallas as pl
from jax.experimental.pallas import tpu as pltpu
from jax.experimental.pallas import tpu_sc as plsc

N = 10000
E = 160000
D = 512
H = 256
HH = H // 2

NC = 2
NS = 16
LANES = 16

R = 10240
ROWS_PER_TILE = R // NS
E_PAD = 163840
EPT = E_PAD // NS
CHUNK = 128
NCHUNK = EPT // CHUNK
SEG = 40
NSEG = NCHUNK // SEG
EPT_DEG = E_PAD // (NC * NS)

_MESH = plsc.VectorSubcoreMesh(
    core_axis_name="c", subcore_axis_name="s", num_cores=NC, num_subcores=NS
)

_SC_PARAMS = pltpu.CompilerParams(needs_layout_passes=False)


def _sc_degrees_body(srcd_hbm, dstd_hbm, dego_hbm, degi_hbm,
                     src_v, dst_v, dego_v, degi_v):
    cid = lax.axis_index("c")
    sid = lax.axis_index("s")
    wid = sid * NC + cid

    pltpu.sync_copy(srcd_hbm.at[wid], src_v)
    pltpu.sync_copy(dstd_hbm.at[wid], dst_v)

    zeros16 = jnp.zeros((LANES,), jnp.float32)

    def zero_body(i, _):
        dego_v[pl.ds(i * LANES, LANES)] = zeros16
        degi_v[pl.ds(i * LANES, LANES)] = zeros16
        return _

    lax.fori_loop(0, R // LANES, zero_body, None)

    ones16 = jnp.ones((LANES,), jnp.float32)

    def hist_body(i, _):
        s_idx = src_v[pl.ds(i * LANES, LANES)]
        plsc.addupdate_scatter(dego_v, [s_idx], ones16)
        d_idx = dst_v[pl.ds(i * LANES, LANES)]
        plsc.addupdate_scatter(degi_v, [d_idx], ones16)
        return _

    lax.fori_loop(0, EPT_DEG // LANES, hist_body, None)

    pltpu.sync_copy(dego_v, dego_hbm.at[wid])
    pltpu.sync_copy(degi_v, degi_hbm.at[wid])


_sc_degrees = pl.kernel(
    _sc_degrees_body,
    out_type=(
        jax.ShapeDtypeStruct((NC * NS, R), jnp.float32),
        jax.ShapeDtypeStruct((NC * NS, R), jnp.float32),
    ),
    mesh=_MESH,
    scratch_types=[
        pltpu.VMEM((EPT_DEG,), jnp.int32),
        pltpu.VMEM((EPT_DEG,), jnp.int32),
        pltpu.VMEM((R,), jnp.float32),
        pltpu.VMEM((R,), jnp.float32),
    ],
    compiler_params=_SC_PARAMS,
)


def _sc_propagate_body(table_hbm, srcp0_hbm, srcp1_hbm, dstp_hbm, agg_hbm,
                       src_v, dst_v, rows_v, rows_v1, acc,
                       sem, sem1, sem2, sem3):
    cid = lax.axis_index("c")
    sid = lax.axis_index("s")

    zeros16 = jnp.zeros((LANES,), jnp.float32)

    def zrow(i, _):
        r = i // (HH // LANES)
        k = i % (HH // LANES)
        rows_v[r, pl.ds(k * LANES, LANES)] = zeros16
        return _

    lax.fori_loop(0, CHUNK * (HH // LANES), zrow, None)

    base = sid * ROWS_PER_TILE
    for z in range(ROWS_PER_TILE // CHUNK):
        pltpu.sync_copy(rows_v, acc.at[pl.ds(base + z * CHUNK, CHUNK)])

    plsc.subcore_barrier()

    def wait_gather(buf, s):
        pltpu.make_async_copy(table_hbm.at[src_v.at[0]], buf, s).wait()

    def wait_scatter(buf, s):
        pltpu.make_async_copy(buf, acc.at[dst_v.at[0]], s).wait()

    def seg_body(sg, _):
        @pl.when(cid == 0)
        def _():
            pltpu.sync_copy(srcp0_hbm.at[sid, sg], src_v)

        @pl.when(cid == 1)
        def _():
            pltpu.sync_copy(srcp1_hbm.at[sid, sg], src_v)

        pltpu.sync_copy(dstp_hbm.at[sid, sg], dst_v)

        pltpu.async_copy(table_hbm.at[src_v.at[0]], rows_v, sem)
        pltpu.async_copy(table_hbm.at[src_v.at[1]], rows_v1, sem1)

        def edge_body(g, _):
            j0 = 2 * g
            wait_gather(rows_v, sem)
            pltpu.async_copy(rows_v, acc.at[dst_v.at[j0]], sem2, add=True)
            wait_gather(rows_v1, sem1)
            pltpu.async_copy(rows_v1, acc.at[dst_v.at[j0 + 1]], sem3, add=True)

            @pl.when(j0 + 2 < SEG)
            def _():
                wait_scatter(rows_v, sem2)
                pltpu.async_copy(table_hbm.at[src_v.at[j0 + 2]], rows_v, sem)

            @pl.when(j0 + 3 < SEG)
            def _():
                wait_scatter(rows_v1, sem3)
                pltpu.async_copy(table_hbm.at[src_v.at[j0 + 3]], rows_v1, sem1)

            return _

        lax.fori_loop(0, SEG // 2, edge_body, None)
        wait_scatter(rows_v, sem2)
        wait_scatter(rows_v1, sem3)
        return _

    lax.fori_loop(0, NSEG, seg_body, None)

    plsc.subcore_barrier()

    out_base = cid * R + base
    pltpu.sync_copy(acc.at[pl.ds(base, ROWS_PER_TILE)],
                    agg_hbm.at[pl.ds(out_base, ROWS_PER_TILE)])


_sc_propagate = pl.kernel(
    _sc_propagate_body,
    out_type=jax.ShapeDtypeStruct((NC * R, HH), jnp.float32),
    mesh=_MESH,
    scratch_types=[
        pltpu.VMEM((SEG, CHUNK), jnp.int32),
        pltpu.VMEM((SEG, CHUNK), jnp.int32),
        pltpu.VMEM((CHUNK, HH), jnp.float32),
        pltpu.VMEM((CHUNK, HH), jnp.float32),
        pltpu.VMEM_SHARED((R, HH), jnp.float32),
        pltpu.SemaphoreType.DMA,
        pltpu.SemaphoreType.DMA,
        pltpu.SemaphoreType.DMA,
        pltpu.SemaphoreType.DMA,
    ],
    compiler_params=_SC_PARAMS,
)


_BM = 1024
_GRID = R // _BM


def _dot(a, b):
    return jnp.dot(a, b, preferred_element_type=jnp.float32,
                   precision=lax.Precision.DEFAULT)


def _tc_finalize_deg_body(dego_ref, degi_ref, dinv_out_ref, dinv_in_ref):
    do = jnp.maximum(jnp.sum(dego_ref[...], axis=0), 1.0)
    di = jnp.maximum(jnp.sum(degi_ref[...], axis=0), 1.0)
    dinv_out_ref[...] = lax.rsqrt(do)[:, None]
    dinv_in_ref[...] = lax.rsqrt(di)[:, None]


def _tc_finalize_deg(dego_p, degi_p):
    return pl.pallas_call(
        _tc_finalize_deg_body,
        out_shape=(
            jax.ShapeDtypeStruct((R, 1), jnp.float32),
            jax.ShapeDtypeStruct((R, 1), jnp.float32),
        ),
    )(dego_p, degi_p)


def _tc_y1_body(x_ref, w_ref, dinv_ref, out_ref):
    y = _dot(x_ref[...], w_ref[...]) * dinv_ref[...]
    out_ref[0, :, :] = y[:, :HH]
    out_ref[1, :, :] = y[:, HH:]


def _tc_y1(feats_p, W_gc1, dinv_out):
    return pl.pallas_call(
        _tc_y1_body,
        grid=(_GRID,),
        in_specs=[
            pl.BlockSpec((_BM, D), lambda r: (r, 0)),
            pl.BlockSpec((D, H), lambda r: (0, 0)),
            pl.BlockSpec((_BM, 1), lambda r: (r, 0)),
        ],
        out_specs=pl.BlockSpec((NC, _BM, HH), lambda r: (0, r, 0)),
        out_shape=jax.ShapeDtypeStruct((NC, R, HH), jnp.float32),
    )(feats_p, W_gc1, dinv_out)


def _tc_mid_body(agg_ref, din_ref, dout_ref, b_ref, out_ref):
    x = jax.nn.relu(agg_ref[...] * din_ref[...][None] + b_ref[...])
    out_ref[...] = x * dout_ref[...][None]


def _tc_mid(agg1, dinv_in, dinv_out, b_gc1_2):
    return pl.pallas_call(
        _tc_mid_body,
        grid=(_GRID,),
        in_specs=[
            pl.BlockSpec((NC, _BM, HH), lambda r: (0, r, 0)),
            pl.BlockSpec((_BM, 1), lambda r: (r, 0)),
            pl.BlockSpec((_BM, 1), lambda r: (r, 0)),
            pl.BlockSpec((NC, 1, HH), lambda r: (0, 0, 0)),
        ],
        out_specs=pl.BlockSpec((NC, _BM, HH), lambda r: (0, r, 0)),
        out_shape=jax.ShapeDtypeStruct((NC, R, HH), jnp.float32),
    )(agg1, dinv_in, dinv_out, b_gc1_2)


def _tc_final_body(agg_ref, din_ref, feat_ref, wg2_ref, bg2_ref,
                   wm1_ref, bm1_ref, wm2_ref, bm2_ref, out_ref):
    a = jnp.concatenate([agg_ref[0], agg_ref[1]], axis=1) * din_ref[...]
    gcn = _dot(a, wg2_ref[...]) + bg2_ref[...] + feat_ref[...]
    m = jax.nn.relu(_dot(gcn, wm1_ref[...]) + bm1_ref[...])
    out_ref[...] = _dot(m, wm2_ref[...]) + bm2_ref[...] + gcn


def _tc_final(agg2, dinv_in, feats_p, W_gc2, b_gc2, W_m1, b_m1, W_m2, b_m2):
    return pl.pallas_call(
        _tc_final_body,
        grid=(_GRID,),
        in_specs=[
            pl.BlockSpec((NC, _BM, HH), lambda r: (0, r, 0)),
            pl.BlockSpec((_BM, 1), lambda r: (r, 0)),
            pl.BlockSpec((_BM, D), lambda r: (r, 0)),
            pl.BlockSpec((H, D), lambda r: (0, 0)),
            pl.BlockSpec((1, D), lambda r: (0, 0)),
            pl.BlockSpec((D, H), lambda r: (0, 0)),
            pl.BlockSpec((1, H), lambda r: (0, 0)),
            pl.BlockSpec((H, D), lambda r: (0, 0)),
            pl.BlockSpec((1, D), lambda r: (0, 0)),
        ],
        out_specs=pl.BlockSpec((_BM, D), lambda r: (r, 0)),
        out_shape=jax.ShapeDtypeStruct((R, D), jnp.float32),
    )(agg2, dinv_in, feats_p, W_gc2, b_gc2, W_m1, b_m1, W_m2, b_m2)


def kernel(features, edge_index, W_gc1, b_gc1, W_gc2, b_gc2,
           W_m1, b_m1, W_m2, b_m2):
    src = edge_index[0]
    dst = edge_index[1]
    pad = jnp.full((E_PAD - E,), N, dtype=jnp.int32)
    src_p = jnp.concatenate([src, pad])
    dst_p = jnp.concatenate([dst, pad])

    srcd = src_p.reshape(NC * NS, EPT_DEG)
    dstd = dst_p.reshape(NC * NS, EPT_DEG)
    srcp0 = src_p.reshape(NS, NSEG, SEG, CHUNK)
    srcp1 = srcp0 + R
    dstp = dst_p.reshape(NS, NSEG, SEG, CHUNK)

    feats_p = jnp.pad(features, ((0, R - N), (0, 0)))

    dego_p, degi_p = _sc_degrees(srcd, dstd)
    dinv_out, dinv_in = _tc_finalize_deg(dego_p, degi_p)

    y1 = _tc_y1(feats_p, W_gc1, dinv_out)
    agg1 = _sc_propagate(y1.reshape(NC * R, HH), srcp0, srcp1, dstp)
    h2 = _tc_mid(agg1.reshape(NC, R, HH), dinv_in, dinv_out,
                 b_gc1.reshape(NC, 1, HH))
    agg2 = _sc_propagate(h2.reshape(NC * R, HH), srcp0, srcp1, dstp)
    out_p = _tc_final(agg2.reshape(NC, R, HH), dinv_in, feats_p,
                      W_gc2, b_gc2.reshape(1, D), W_m1, b_m1.reshape(1, H),
                      W_m2, b_m2.reshape(1, D))
    return out_p[:N]
